# baseline (speedup 1.0000x reference)
import jax
import jax.numpy as jnp
from jax import lax
from jax.experimental import pallas as pl
from jax.experimental.pallas import tpu as pltpu
from jax.experimental.pallas import tpu_sc as plsc

_BATCH = 128
_NUM_QUERIES = 1024
_QUERIES_DIM = 512
_NC = 2
_NS = 16
_NW = _NC * _NS
_ROWS_PER_W = _NUM_QUERIES // _NW


def _sc_body(table_hbm, out_hbm, rows_v, sem):
    wid = lax.axis_index("s") * _NC + lax.axis_index("c")
    base = wid * _ROWS_PER_W
    pltpu.sync_copy(table_hbm.at[pl.ds(base, _ROWS_PER_W)], rows_v)
    copies = []
    for b in range(_BATCH):
        copies.append(
            pltpu.async_copy(
                rows_v,
                out_hbm.at[pl.ds(b * _NUM_QUERIES + base, _ROWS_PER_W)],
                sem,
            )
        )
    for c in copies:
        c.wait()


def kernel(queries_weight, batch_size, num_queries):
    del batch_size, num_queries
    out2d = pl.kernel(
        _sc_body,
        out_type=jax.ShapeDtypeStruct(
            (_BATCH * _NUM_QUERIES, _QUERIES_DIM), jnp.float32
        ),
        mesh=plsc.VectorSubcoreMesh(core_axis_name="c", subcore_axis_name="s"),
        scratch_types=[
            pltpu.VMEM((_ROWS_PER_W, _QUERIES_DIM), jnp.float32),
            pltpu.SemaphoreType.DMA,
        ],
    )(queries_weight)
    return out2d.reshape(_BATCH, _NUM_QUERIES, _QUERIES_DIM)


# speedup vs baseline: 1.2681x; 1.2681x over previous
import jax
import jax.numpy as jnp
from jax.experimental import pallas as pl
from jax.experimental.pallas import tpu as pltpu

_BATCH = 128
_NUM_QUERIES = 1024
_QUERIES_DIM = 512
_DUP = 2
_NSEM = 8


def _body(w_hbm, o_hbm, w_vmem, in_sems, out_sems):
    loads = [
        pltpu.make_async_copy(w_hbm, w_vmem.at[d], in_sems.at[d])
        for d in range(_DUP)
    ]
    for ld in loads:
        ld.start()
    for ld in loads:
        ld.wait()
    nsteps = _BATCH // _DUP
    for i in range(nsteps):
        if i >= _NSEM:
            pltpu.make_async_copy(
                w_vmem,
                o_hbm.at[pl.ds((i - _NSEM) * _DUP, _DUP)],
                out_sems.at[(i - _NSEM) % _NSEM],
            ).wait()
        pltpu.make_async_copy(
            w_vmem, o_hbm.at[pl.ds(i * _DUP, _DUP)], out_sems.at[i % _NSEM]
        ).start()
    for i in range(nsteps - _NSEM, nsteps):
        pltpu.make_async_copy(
            w_vmem, o_hbm.at[pl.ds(i * _DUP, _DUP)], out_sems.at[i % _NSEM]
        ).wait()


def kernel(queries_weight, batch_size, num_queries):
    del batch_size, num_queries
    return pl.pallas_call(
        _body,
        in_specs=[pl.BlockSpec(memory_space=pltpu.MemorySpace.HBM)],
        out_specs=pl.BlockSpec(memory_space=pltpu.MemorySpace.HBM),
        out_shape=jax.ShapeDtypeStruct(
            (_BATCH, _NUM_QUERIES, _QUERIES_DIM), queries_weight.dtype
        ),
        scratch_shapes=[
            pltpu.VMEM((_DUP, _NUM_QUERIES, _QUERIES_DIM), jnp.float32),
            pltpu.SemaphoreType.DMA((_DUP,)),
            pltpu.SemaphoreType.DMA((_NSEM,)),
        ],
    )(queries_weight)


# speedup vs baseline: 1.2945x; 1.0208x over previous
import jax
import jax.numpy as jnp
from jax.experimental import pallas as pl

_BATCH = 128
_NUM_QUERIES = 1024
_QUERIES_DIM = 512
_B_BLK = 2


def _broadcast_body(w_ref, o_ref):
    o_ref[...] = jnp.broadcast_to(w_ref[...][None], o_ref.shape)


def kernel(queries_weight, batch_size, num_queries):
    del batch_size, num_queries
    return pl.pallas_call(
        _broadcast_body,
        grid=(_BATCH // _B_BLK,),
        in_specs=[
            pl.BlockSpec((_NUM_QUERIES, _QUERIES_DIM), lambda i: (0, 0)),
        ],
        out_specs=pl.BlockSpec(
            (_B_BLK, _NUM_QUERIES, _QUERIES_DIM), lambda i: (i, 0, 0)
        ),
        out_shape=jax.ShapeDtypeStruct(
            (_BATCH, _NUM_QUERIES, _QUERIES_DIM), queries_weight.dtype
        ),
    )(queries_weight)
